# baseline (speedup 1.0000x reference)
import functools

import jax
import jax.numpy as jnp
from jax import lax
from jax.experimental import pallas as pl
from jax.experimental.pallas import tpu as pltpu
from jax.experimental.pallas import tpu_sc as plsc

VOCAB = 100000
D = 1024
NEW_ROWS = 702
TOTAL = 8192
MAX_IDX = VOCAB + NEW_ROWS - 2

NW = 32
B_PER_W = TOTAL // NW
CHUNK = 16
NCHUNK = B_PER_W // CHUNK
NGROUP = B_PER_W // 16
GPC = CHUNK // 16
DBLK = D // 16
NBUF = 4


def _body(x_hbm, orig_hbm, new_hbm, out_hbm,
          idx_v, lo_v, hi_v, buf0, buf1, buf2, buf3, fbuf,
          g_s0, g_s1, g_s2, g_s3, w_s0, w_s1, w_s2, w_s3, f_s):
    wid = lax.axis_index("s") * 2 + lax.axis_index("c")
    base = wid * B_PER_W

    pltpu.sync_copy(x_hbm.at[pl.ds(base, B_PER_W)], idx_v)

    cnts = []
    for i in range(NGROUP):
        v = idx_v[pl.ds(i * 16, 16)]
        v = jnp.maximum(v, 0)
        vc = jnp.minimum(v, MAX_IDX)
        is_hi = v >= VOCAB
        hi = jnp.where(is_hi, vc - (VOCAB - 1), 0)
        lo_v[pl.ds(i * 16, 16)] = jnp.minimum(v, VOCAB - 1)
        hi_v[i, :] = hi
        acc = hi[0]
        for t in range(1, 16):
            acc = acc + hi[t]
        cnts.append(acc)

    bufs = (buf0, buf1, buf2, buf3)
    g_sem = (g_s0, g_s1, g_s2, g_s3)
    w_sem = (w_s0, w_s1, w_s2, w_s3)
    gath = [None] * NBUF
    fired = [False] * NBUF

    def _gath(j):
        return pltpu.async_copy(
            orig_hbm.at[lo_v.at[pl.ds(j * CHUNK, CHUNK)]],
            bufs[j % NBUF], g_sem[j % NBUF])

    AHEAD = NBUF - 2
    for j in range(min(AHEAD, NCHUNK)):
        gath[j % NBUF] = _gath(j)
    for j in range(NCHUNK):
        s = j % NBUF
        if j + AHEAD < NCHUNK:
            o = (j + AHEAD) % NBUF
            if fired[o]:
                pltpu.make_async_copy(
                    bufs[o],
                    out_hbm.at[pl.ds(base + (j + AHEAD - NBUF) * CHUNK,
                                     CHUNK)],
                    w_sem[o]).wait()
                fired[o] = False
            gath[o] = _gath(j + AHEAD)
        gath[s].wait()

        for g in range(GPC):
            i = GPC * j + g

            @pl.when(cnts[i] > 0)
            def _patch(s=s, g=g, i=i):
                pltpu.async_copy(new_hbm.at[hi_v.at[i]], fbuf, f_s).wait()
                hv = hi_v[i, :]
                for t in range(16):
                    @pl.when(hv[t] > 0)
                    def _lane(s=s, g=g, t=t):
                        def _cp(q, _):
                            col = q * 16
                            bufs[s][g * 16 + t, pl.ds(col, 16)] = (
                                fbuf[t, pl.ds(col, 16)])
                            return 0
                        lax.fori_loop(0, DBLK, _cp, 0)

        pltpu.async_copy(bufs[s],
                         out_hbm.at[pl.ds(base + j * CHUNK, CHUNK)],
                         w_sem[s])
        fired[s] = True
    for s in range(NBUF):
        if fired[s]:
            pltpu.make_async_copy(
                bufs[s], out_hbm.at[pl.ds(base, CHUNK)], w_sem[s]).wait()


@jax.jit
def _gather(x_flat, original_weight, new_weight):
    mesh = plsc.VectorSubcoreMesh(core_axis_name="c", subcore_axis_name="s")
    k = functools.partial(
        pl.kernel,
        mesh=mesh,
        out_type=jax.ShapeDtypeStruct((TOTAL, D), jnp.float32),
        scratch_types=[
            pltpu.VMEM((B_PER_W,), jnp.int32),
            pltpu.VMEM((B_PER_W,), jnp.int32),
            pltpu.VMEM((NGROUP, 16), jnp.int32),
            pltpu.VMEM((CHUNK, D), jnp.float32),
            pltpu.VMEM((CHUNK, D), jnp.float32),
            pltpu.VMEM((CHUNK, D), jnp.float32),
            pltpu.VMEM((CHUNK, D), jnp.float32),
            pltpu.VMEM((16, D), jnp.float32),
        ] + [pltpu.SemaphoreType.DMA] * 9,
    )(_body)
    return k(x_flat, original_weight, new_weight)


def kernel(x, original_weight, new_weight):
    out = _gather(x.reshape(-1), original_weight, new_weight)
    return out.reshape(*x.shape, D)
